# baseline (speedup 1.0000x reference)
import functools

import jax
import jax.numpy as jnp
from jax import lax
from jax.experimental import pallas as pl
from jax.experimental.pallas import tpu as pltpu
from jax.experimental.pallas import tpu_sc as plsc

NOP = 10000
NRES = 1000
E = 160000
EMB = 8

NC = 2
NS = 16
NW = NC * NS
CH = 128
EP = 163840
NCH = EP // (NW * CH)
PW = NCH * CH
R_OP = 10240
R_RES = 1024

_MESH = plsc.VectorSubcoreMesh(core_axis_name="c", subcore_axis_name="s")


def _pad_idx(idx, fill):
    pad = jnp.full((EP - E,), fill, jnp.int32)
    return jnp.concatenate([idx.astype(jnp.int32), pad]).reshape(NW, NCH, CH)


def _sc_aggregate(table16, gidx, sidx, zrows, nrows):
    rpt = nrows // NS

    def body(table_h, gidx_h, sidx_h, zeros_h, out_h, gidx_v, sidx_v, rows_v, acc_sh, sem):
        c = lax.axis_index("c")
        s = lax.axis_index("s")
        wid = s * NC + c
        pltpu.sync_copy(zeros_h.at[pl.ds(s * rpt, rpt)], acc_sh.at[pl.ds(s * rpt, rpt)])
        pltpu.sync_copy(gidx_h.at[wid], gidx_v)
        pltpu.sync_copy(sidx_h.at[wid], sidx_v)
        plsc.subcore_barrier()

        @pl.loop(0, NCH)
        def _fire(j):
            pltpu.async_copy(table_h.at[gidx_v.at[j]], rows_v.at[j], sem)

        @pl.loop(0, NCH)
        def _drain(j):
            pltpu.make_async_copy(table_h.at[gidx_v.at[j]], rows_v.at[j], sem).wait()
            pltpu.sync_copy(rows_v.at[j], acc_sh.at[sidx_v.at[j]], add=True)

        plsc.subcore_barrier()
        pltpu.sync_copy(acc_sh.at[pl.ds(s * rpt, rpt)], out_h.at[c, pl.ds(s * rpt, rpt)])

    f = pl.kernel(
        body,
        out_type=jax.ShapeDtypeStruct((NC, nrows, 16), jnp.float32),
        mesh=_MESH,
        compiler_params=pltpu.CompilerParams(use_tc_tiling_on_sc=False),
        scratch_types=[
            pltpu.VMEM((NCH, CH), jnp.int32),
            pltpu.VMEM((NCH, CH), jnp.int32),
            pltpu.VMEM((NCH, CH, 16), jnp.float32),
            pltpu.VMEM_SHARED((nrows, 16), jnp.float32),
            pltpu.SemaphoreType.DMA,
        ],
    )
    return f(table16, gidx, sidx, zrows)


def _sc_gather2(p8, q8, sidx, didx):
    def body(p_h, q_h, si_h, di_h, op_h, oq_h, si_v, di_v, rp_v, rq_v, semp, semq):
        c = lax.axis_index("c")
        s = lax.axis_index("s")
        wid = s * NC + c
        pltpu.sync_copy(si_h.at[wid], si_v)
        pltpu.sync_copy(di_h.at[wid], di_v)

        @pl.loop(0, NCH)
        def _fire(j):
            pltpu.async_copy(p_h.at[si_v.at[j]], rp_v.at[pl.ds(j * CH, CH)], semp)
            pltpu.async_copy(q_h.at[di_v.at[j]], rq_v.at[pl.ds(j * CH, CH)], semq)

        @pl.loop(0, NCH)
        def _drain(j):
            pltpu.make_async_copy(p_h.at[si_v.at[j]], rp_v.at[pl.ds(j * CH, CH)], semp).wait()
            pltpu.make_async_copy(q_h.at[di_v.at[j]], rq_v.at[pl.ds(j * CH, CH)], semq).wait()

        pltpu.sync_copy(rp_v, op_h.at[pl.ds(wid * PW, PW)])
        pltpu.sync_copy(rq_v, oq_h.at[pl.ds(wid * PW, PW)])

    f = pl.kernel(
        body,
        out_type=[
            jax.ShapeDtypeStruct((EP, EMB), jnp.float32),
            jax.ShapeDtypeStruct((EP, EMB), jnp.float32),
        ],
        mesh=_MESH,
        compiler_params=pltpu.CompilerParams(use_tc_tiling_on_sc=False),
        scratch_types=[
            pltpu.VMEM((NCH, CH), jnp.int32),
            pltpu.VMEM((NCH, CH), jnp.int32),
            pltpu.VMEM((PW, EMB), jnp.float32),
            pltpu.VMEM((PW, EMB), jnp.float32),
            pltpu.SemaphoreType.DMA,
            pltpu.SemaphoreType.DMA,
        ],
    )
    return f(p8, q8, sidx, didx)


def _sc_scatter(m16, sidx, zrows, nrows):
    rpt = nrows // NS

    def body(m_h, sidx_h, zeros_h, out_h, sidx_v, rows_v, acc_sh):
        c = lax.axis_index("c")
        s = lax.axis_index("s")
        wid = s * NC + c
        pltpu.sync_copy(zeros_h.at[pl.ds(s * rpt, rpt)], acc_sh.at[pl.ds(s * rpt, rpt)])
        pltpu.sync_copy(sidx_h.at[wid], sidx_v)
        pltpu.sync_copy(m_h.at[pl.ds(wid * PW, PW)], rows_v)
        plsc.subcore_barrier()

        @pl.loop(0, NCH)
        def _scat(j):
            pltpu.sync_copy(rows_v.at[pl.ds(j * CH, CH)], acc_sh.at[sidx_v.at[j]], add=True)

        plsc.subcore_barrier()
        pltpu.sync_copy(acc_sh.at[pl.ds(s * rpt, rpt)], out_h.at[c, pl.ds(s * rpt, rpt)])

    f = pl.kernel(
        body,
        out_type=jax.ShapeDtypeStruct((NC, nrows, 16), jnp.float32),
        mesh=_MESH,
        compiler_params=pltpu.CompilerParams(use_tc_tiling_on_sc=False),
        scratch_types=[
            pltpu.VMEM((NCH, CH), jnp.int32),
            pltpu.VMEM((PW, 16), jnp.float32),
            pltpu.VMEM_SHARED((nrows, 16), jnp.float32),
        ],
    )
    return f(m16, sidx, zrows)


def _dot(a, b):
    return jnp.dot(a.astype(jnp.bfloat16), b.astype(jnp.bfloat16),
                   preferred_element_type=jnp.float32)


def _elu(x):
    return jnp.where(x > 0, x, jnp.exp(jnp.minimum(x, 0.0)) - 1.0)


def _mlp3(x, w0, b0, w1, b1, w2, b2):
    h = _elu(_dot(x, w0[...]) + b0[...])
    h = _elu(_dot(h, w1[...]) + b1[...])
    return _dot(h, w2[...]) + b2[...]


def _with_count_col(t, count_val):
    n = t.shape[0]
    col = lax.broadcasted_iota(jnp.int32, (n, 16), 1)
    tt = jnp.concatenate([t, t], axis=1)
    return jnp.where(col < EMB, tt, jnp.where(col == EMB, count_val, 0.0))


def _mlp_flat(mlp):
    out = []
    for lin in mlp:
        out.append(lin["W"])
        out.append(lin["b"].reshape(1, -1))
    return out


def _full_spec(a):
    return pl.BlockSpec(a.shape, lambda *_: (0,) * a.ndim)


def _tc_res_pre(x, w, b):
    def body(x_ref, w_ref, b_ref, out_ref):
        t = _dot(x_ref[...], w_ref[...]) + b_ref[...]
        out_ref[...] = _with_count_col(t, 1.0)

    return pl.pallas_call(
        body,
        out_shape=jax.ShapeDtypeStruct((NRES, 16), jnp.float32),
    )(x, w, b.reshape(1, -1))


def _tc_res_next(parts, w, b):
    def body(p_ref, w_ref, b_ref, out_ref):
        p = p_ref[...]
        sums = (p[0] + p[1])[:NRES]
        r = sums[:, :EMB] / jnp.maximum(sums[:, EMB:EMB + 1], 1.0)
        t = _dot(r, w_ref[...]) + b_ref[...]
        out_ref[...] = _with_count_col(t, 1.0)

    return pl.pallas_call(
        body,
        out_shape=jax.ShapeDtypeStruct((NRES, 16), jnp.float32),
    )(parts, w, b.reshape(1, -1))


def _tc_res_fin(parts):
    def body(p_ref, r_ref, r16_ref):
        p = p_ref[...]
        sums = (p[0] + p[1])[:NRES]
        r = sums[:, :EMB] / jnp.maximum(sums[:, EMB:EMB + 1], 1.0)
        r_ref[...] = r
        r16_ref[...] = _with_count_col(r, 0.0)

    return pl.pallas_call(
        body,
        out_shape=[
            jax.ShapeDtypeStruct((NRES, EMB), jnp.float32),
            jax.ShapeDtypeStruct((NRES, 16), jnp.float32),
        ],
    )(parts)


_NBLK = 1000


def _tc_node(x, aggparts, lp):
    fi = x.shape[1]
    grid = NOP // _NBLK
    weights = (_mlp_flat(lp["predecessor"]) + _mlp_flat(lp["successor"])
               + _mlp_flat(lp["resources"]) + _mlp_flat(lp["same"]))

    def body(x_ref, agg_ref, *refs):
        w = refs[:24]
        p_ref, q_ref, s2_ref = refs[24:]
        x_v = x_ref[...]
        a = agg_ref[...]
        aggv = (a[0] + a[1])[:, :EMB]
        p_ref[...] = _mlp3(x_v, *w[0:6])
        q_ref[...] = _mlp3(aggv, *w[12:18]) + _mlp3(x_v, *w[6:12])
        s2_ref[...] = _mlp3(x_v, *w[18:24])

    in_specs = [
        pl.BlockSpec((_NBLK, fi), lambda i: (i, 0)),
        pl.BlockSpec((NC, _NBLK, 16), lambda i: (0, i, 0)),
    ] + [_full_spec(a) for a in weights]
    out_spec = pl.BlockSpec((_NBLK, EMB), lambda i: (i, 0))
    return pl.pallas_call(
        body,
        grid=(grid,),
        in_specs=in_specs,
        out_specs=[out_spec] * 3,
        out_shape=[jax.ShapeDtypeStruct((NOP, EMB), jnp.float32)] * 3,
    )(x, aggparts, *weights)


_EBLK = 4096


def _tc_comb(x128, mlp):
    grid = EP // _EBLK

    w1, b1 = mlp[0]["W"], mlp[0]["b"]
    w2, b2 = mlp[1]["W"], mlp[1]["b"]
    w3, b3 = mlp[2]["W"], mlp[2]["b"]
    w1bd = jnp.zeros((128, 16 * 128), jnp.float32)
    for g in range(16):
        w1bd = w1bd.at[g * EMB:(g + 1) * EMB, g * 128:(g + 1) * 128].set(w1)
    b1t = jnp.tile(b1, 16).reshape(1, 16 * 128)
    w3e = jnp.concatenate([w3, jnp.zeros((128, 8), jnp.float32)], axis=1)
    w3bd = jnp.zeros((8 * 128, 128), jnp.float32)
    for g in range(8):
        w3bd = w3bd.at[g * 128:(g + 1) * 128, g * 16:(g + 1) * 16].set(w3e)
    b3e = jnp.concatenate([b3, jnp.ones((1,), jnp.float32),
                           jnp.zeros((7,), jnp.float32)])
    b3t = jnp.tile(b3e, 8).reshape(1, 8 * 16)

    def body(x_ref, w1_ref, b1_ref, w2_ref, b2_ref, w3_ref, b3_ref, out_ref):
        h1 = _elu(_dot(x_ref[...], w1_ref[...]) + b1_ref[...])
        h1 = h1.reshape(_EBLK, 128)
        h2 = _elu(_dot(h1, w2_ref[...]) + b2_ref[...])
        h2 = h2.reshape(_EBLK // 8, 8 * 128)
        out_ref[...] = _dot(h2, w3_ref[...]) + b3_ref[...]

    args = [w1bd, b1t, w2, b2.reshape(1, -1), w3bd, b3t]
    return pl.pallas_call(
        body,
        grid=(grid,),
        in_specs=[pl.BlockSpec((_EBLK // 16, 128), lambda i: (i, 0))]
        + [_full_spec(a) for a in args],
        out_specs=pl.BlockSpec((_EBLK // 8, 128), lambda i: (i, 0)),
        out_shape=jax.ShapeDtypeStruct((EP // 8, 128), jnp.float32),
    )(x128, *args)


def _tc_fin(parts, s2):
    grid = NOP // _NBLK

    def body(p_ref, s2_ref, o_ref):
        p = p_ref[...]
        tot = p[0] + p[1]
        o_ref[...] = tot[:, :EMB] + tot[:, EMB:EMB + 1] * s2_ref[...]

    return pl.pallas_call(
        body,
        grid=(grid,),
        in_specs=[
            pl.BlockSpec((NC, _NBLK, 16), lambda i: (0, i, 0)),
            pl.BlockSpec((_NBLK, EMB), lambda i: (i, 0)),
        ],
        out_specs=pl.BlockSpec((_NBLK, EMB), lambda i: (i, 0)),
        out_shape=jax.ShapeDtypeStruct((NOP, EMB), jnp.float32),
    )(parts, s2)


def kernel(x_op, x_res, params, precedence_edges, requirement_edges):
    rq_src = requirement_edges[0]
    rq_dst = requirement_edges[1]
    pe_src = precedence_edges[0]
    pe_dst = precedence_edges[1]

    g_rq_src = _pad_idx(rq_src, 0)
    s_rq_dst = _pad_idx(rq_dst, NRES)
    g_rq_dst = _pad_idx(rq_dst, 0)
    s_rq_src = _pad_idx(rq_src, NOP)
    g_pe_src = _pad_idx(pe_src, 0)
    g_pe_dst = _pad_idx(pe_dst, 0)
    s_pe_dst = _pad_idx(pe_dst, NOP)

    z_res = jnp.zeros((R_RES, 16), jnp.float32)
    z_op = jnp.zeros((R_OP, 16), jnp.float32)

    lp0, lp1 = params["res_layers"]
    t16 = _tc_res_pre(x_res, lp0["W"], lp0["b"])
    parts = _sc_aggregate(t16, g_rq_src, s_rq_dst, z_res, R_RES)
    t16 = _tc_res_next(parts, lp1["W"], lp1["b"])
    parts = _sc_aggregate(t16, g_rq_src, s_rq_dst, z_res, R_RES)
    r, r16 = _tc_res_fin(parts)

    aggparts = _sc_aggregate(r16, g_rq_dst, s_rq_src, z_op, R_OP)

    o = x_op
    for lp in params["op_layers"]:
        p8, q8, s2 = _tc_node(o, aggparts, lp)
        pg, qg = _sc_gather2(p8, q8, g_pe_src, g_pe_dst)
        x128 = (pg + qg).reshape(EP // 16, 128)
        m16 = _tc_comb(x128, lp["combined"])
        eparts = _sc_scatter(m16.reshape(EP, 16), s_pe_dst, z_op, R_OP)
        o = _tc_fin(eparts, s2)

    return o, r


# speedup vs baseline: 1.5775x; 1.5775x over previous
import functools

import jax
import jax.numpy as jnp
from jax import lax
from jax.experimental import pallas as pl
from jax.experimental.pallas import tpu as pltpu
from jax.experimental.pallas import tpu_sc as plsc

NOP = 10000
NRES = 1000
E = 160000
EMB = 8

NC = 2
NS = 16
NW = NC * NS
CH = 128
EP = 163840
NCH = EP // (NW * CH)
PW = NCH * CH
R_OP = 10240
R_RES = 1024

_MESH = plsc.VectorSubcoreMesh(core_axis_name="c", subcore_axis_name="s")


def _pad_idx(idx, fill):
    pad = jnp.full((EP - E,), fill, jnp.int32)
    return jnp.concatenate([idx.astype(jnp.int32), pad]).reshape(NW, NCH, CH)


def _sc_aggregate(table16, gidx, sidx, zrows, nrows):
    rpt = nrows // NS

    def body(table_h, gidx_h, sidx_h, zeros_h, out_h, gidx_v, sidx_v, rows_v, acc_sh, sem):
        c = lax.axis_index("c")
        s = lax.axis_index("s")
        wid = s * NC + c
        pltpu.sync_copy(zeros_h.at[pl.ds(s * rpt, rpt)], acc_sh.at[pl.ds(s * rpt, rpt)])
        pltpu.sync_copy(gidx_h.at[wid], gidx_v)
        pltpu.sync_copy(sidx_h.at[wid], sidx_v)
        plsc.subcore_barrier()

        @pl.loop(0, NCH)
        def _fire(j):
            pltpu.async_copy(table_h.at[gidx_v.at[j]], rows_v.at[j], sem)

        @pl.loop(0, NCH)
        def _drain(j):
            pltpu.make_async_copy(table_h.at[gidx_v.at[j]], rows_v.at[j], sem).wait()
            pltpu.sync_copy(rows_v.at[j], acc_sh.at[sidx_v.at[j]], add=True)

        plsc.subcore_barrier()
        pltpu.sync_copy(acc_sh.at[pl.ds(s * rpt, rpt)], out_h.at[c, pl.ds(s * rpt, rpt)])

    f = pl.kernel(
        body,
        out_type=jax.ShapeDtypeStruct((NC, nrows, 16), jnp.float32),
        mesh=_MESH,
        compiler_params=pltpu.CompilerParams(use_tc_tiling_on_sc=False),
        scratch_types=[
            pltpu.VMEM((NCH, CH), jnp.int32),
            pltpu.VMEM((NCH, CH), jnp.int32),
            pltpu.VMEM((NCH, CH, 16), jnp.float32),
            pltpu.VMEM_SHARED((nrows, 16), jnp.float32),
            pltpu.SemaphoreType.DMA,
        ],
    )
    return f(table16, gidx, sidx, zrows)


def _sc_gather2(p8, q8, sidx, didx):
    def body(p_h, q_h, si_h, di_h, op_h, oq_h, si_v, di_v, rp_v, rq_v, semp, semq):
        c = lax.axis_index("c")
        s = lax.axis_index("s")
        wid = s * NC + c
        pltpu.sync_copy(si_h.at[wid], si_v)
        pltpu.sync_copy(di_h.at[wid], di_v)

        @pl.loop(0, NCH)
        def _fire(j):
            pltpu.async_copy(p_h.at[si_v.at[j]], rp_v.at[pl.ds(j * CH, CH)], semp)
            pltpu.async_copy(q_h.at[di_v.at[j]], rq_v.at[pl.ds(j * CH, CH)], semq)

        @pl.loop(0, NCH)
        def _drain(j):
            pltpu.make_async_copy(p_h.at[si_v.at[j]], rp_v.at[pl.ds(j * CH, CH)], semp).wait()
            pltpu.make_async_copy(q_h.at[di_v.at[j]], rq_v.at[pl.ds(j * CH, CH)], semq).wait()

        pltpu.sync_copy(rp_v, op_h.at[pl.ds(wid * PW, PW)])
        pltpu.sync_copy(rq_v, oq_h.at[pl.ds(wid * PW, PW)])

    f = pl.kernel(
        body,
        out_type=[
            jax.ShapeDtypeStruct((EP, EMB), jnp.float32),
            jax.ShapeDtypeStruct((EP, EMB), jnp.float32),
        ],
        mesh=_MESH,
        compiler_params=pltpu.CompilerParams(use_tc_tiling_on_sc=False),
        scratch_types=[
            pltpu.VMEM((NCH, CH), jnp.int32),
            pltpu.VMEM((NCH, CH), jnp.int32),
            pltpu.VMEM((PW, EMB), jnp.float32),
            pltpu.VMEM((PW, EMB), jnp.float32),
            pltpu.SemaphoreType.DMA,
            pltpu.SemaphoreType.DMA,
        ],
    )
    return f(p8, q8, sidx, didx)


def _sc_scatter(m16, sidx, zrows, nrows):
    rpt = nrows // NS

    def body(m_h, sidx_h, zeros_h, out_h, sidx_v, rows_v, acc_sh):
        c = lax.axis_index("c")
        s = lax.axis_index("s")
        wid = s * NC + c
        pltpu.sync_copy(zeros_h.at[pl.ds(s * rpt, rpt)], acc_sh.at[pl.ds(s * rpt, rpt)])
        pltpu.sync_copy(sidx_h.at[wid], sidx_v)
        pltpu.sync_copy(m_h.at[pl.ds(wid * PW, PW)], rows_v)
        plsc.subcore_barrier()

        @pl.loop(0, NCH)
        def _scat(j):
            pltpu.sync_copy(rows_v.at[pl.ds(j * CH, CH)], acc_sh.at[sidx_v.at[j]], add=True)

        plsc.subcore_barrier()
        pltpu.sync_copy(acc_sh.at[pl.ds(s * rpt, rpt)], out_h.at[c, pl.ds(s * rpt, rpt)])

    f = pl.kernel(
        body,
        out_type=jax.ShapeDtypeStruct((NC, nrows, 16), jnp.float32),
        mesh=_MESH,
        compiler_params=pltpu.CompilerParams(use_tc_tiling_on_sc=False),
        scratch_types=[
            pltpu.VMEM((NCH, CH), jnp.int32),
            pltpu.VMEM((PW, 16), jnp.float32),
            pltpu.VMEM_SHARED((nrows, 16), jnp.float32),
        ],
    )
    return f(m16, sidx, zrows)


def _dot(a, b):
    return jnp.dot(a.astype(jnp.bfloat16), b.astype(jnp.bfloat16),
                   preferred_element_type=jnp.float32)


def _elu(x):
    return jnp.where(x > 0, x, jnp.exp(jnp.minimum(x, 0.0)) - 1.0)


def _mlp3(x, w0, b0, w1, b1, w2, b2):
    h = _elu(_dot(x, w0[...]) + b0[...])
    h = _elu(_dot(h, w1[...]) + b1[...])
    return _dot(h, w2[...]) + b2[...]


def _with_count_col(t, count_val):
    n = t.shape[0]
    col = lax.broadcasted_iota(jnp.int32, (n, 16), 1)
    tt = jnp.concatenate([t, t], axis=1)
    return jnp.where(col < EMB, tt, jnp.where(col == EMB, count_val, 0.0))


def _mlp_flat(mlp):
    out = []
    for lin in mlp:
        out.append(lin["W"])
        out.append(lin["b"].reshape(1, -1))
    return out


def _full_spec(a):
    return pl.BlockSpec(a.shape, lambda *_: (0,) * a.ndim)


def _tc_res_pre(x, w, b):
    def body(x_ref, w_ref, b_ref, out_ref):
        t = _dot(x_ref[...], w_ref[...]) + b_ref[...]
        out_ref[...] = _with_count_col(t, 1.0)

    return pl.pallas_call(
        body,
        out_shape=jax.ShapeDtypeStruct((NRES, 16), jnp.float32),
    )(x, w, b.reshape(1, -1))


def _tc_res_next(parts, w, b):
    def body(p_ref, w_ref, b_ref, out_ref):
        p = p_ref[...]
        sums = (p[0] + p[1])[:NRES]
        r = sums[:, :EMB] / jnp.maximum(sums[:, EMB:EMB + 1], 1.0)
        t = _dot(r, w_ref[...]) + b_ref[...]
        out_ref[...] = _with_count_col(t, 1.0)

    return pl.pallas_call(
        body,
        out_shape=jax.ShapeDtypeStruct((NRES, 16), jnp.float32),
    )(parts, w, b.reshape(1, -1))


def _tc_res_fin(parts):
    def body(p_ref, r_ref, r16_ref):
        p = p_ref[...]
        sums = (p[0] + p[1])[:NRES]
        r = sums[:, :EMB] / jnp.maximum(sums[:, EMB:EMB + 1], 1.0)
        r_ref[...] = r
        r16_ref[...] = _with_count_col(r, 0.0)

    return pl.pallas_call(
        body,
        out_shape=[
            jax.ShapeDtypeStruct((NRES, EMB), jnp.float32),
            jax.ShapeDtypeStruct((NRES, 16), jnp.float32),
        ],
    )(parts)


_NBLK = 1000


def _tc_node(x, aggparts, lp):
    fi = x.shape[1]
    grid = NOP // _NBLK
    weights = (_mlp_flat(lp["predecessor"]) + _mlp_flat(lp["successor"])
               + _mlp_flat(lp["resources"]) + _mlp_flat(lp["same"]))

    def body(x_ref, agg_ref, *refs):
        w = refs[:24]
        p_ref, q_ref, s2_ref = refs[24:]
        x_v = x_ref[...]
        a = agg_ref[...]
        aggv = (a[0] + a[1])[:, :EMB]
        p_ref[...] = _mlp3(x_v, *w[0:6])
        q_ref[...] = _mlp3(aggv, *w[12:18]) + _mlp3(x_v, *w[6:12])
        s2_ref[...] = _mlp3(x_v, *w[18:24])

    in_specs = [
        pl.BlockSpec((_NBLK, fi), lambda i: (i, 0)),
        pl.BlockSpec((NC, _NBLK, 16), lambda i: (0, i, 0)),
    ] + [_full_spec(a) for a in weights]
    out_spec = pl.BlockSpec((_NBLK, EMB), lambda i: (i, 0))
    return pl.pallas_call(
        body,
        grid=(grid,),
        in_specs=in_specs,
        out_specs=[out_spec] * 3,
        out_shape=[jax.ShapeDtypeStruct((NOP, EMB), jnp.float32)] * 3,
    )(x, aggparts, *weights)


_EBLK = 4096


def _tc_comb(pg128, qg128, mlp):
    grid = EP // _EBLK

    w1, b1 = mlp[0]["W"], mlp[0]["b"]
    w2, b2 = mlp[1]["W"], mlp[1]["b"]
    w3, b3 = mlp[2]["W"], mlp[2]["b"]
    w1bd = jnp.zeros((128, 16 * 128), jnp.float32)
    for g in range(16):
        w1bd = w1bd.at[g * EMB:(g + 1) * EMB, g * 128:(g + 1) * 128].set(w1)
    b1t = jnp.tile(b1, 16).reshape(1, 16 * 128)

    def body(pg_ref, qg_ref, w1_ref, b1_ref, w2_ref, b2_ref, w3_ref, b3_ref, out_ref):
        h1 = _elu(_dot(pg_ref[...], w1_ref[...]) + _dot(qg_ref[...], w1_ref[...])
                  + b1_ref[...])
        h1 = h1.reshape(_EBLK, 128)
        h2 = _elu(_dot(h1, w2_ref[...]) + b2_ref[...])
        m = _dot(h2, w3_ref[...]) + b3_ref[...]
        out_ref[...] = _with_count_col(m, 1.0)

    args = [w1bd, b1t, w2, b2.reshape(1, -1), w3, b3.reshape(1, -1)]
    return pl.pallas_call(
        body,
        grid=(grid,),
        in_specs=[pl.BlockSpec((_EBLK // 16, 128), lambda i: (i, 0)),
                  pl.BlockSpec((_EBLK // 16, 128), lambda i: (i, 0))]
        + [_full_spec(a) for a in args],
        out_specs=pl.BlockSpec((_EBLK, 16), lambda i: (i, 0)),
        out_shape=jax.ShapeDtypeStruct((EP, 16), jnp.float32),
    )(pg128, qg128, *args)


def _tc_fin(parts, s2):
    grid = NOP // _NBLK

    def body(p_ref, s2_ref, o_ref):
        p = p_ref[...]
        tot = p[0] + p[1]
        o_ref[...] = tot[:, :EMB] + tot[:, EMB:EMB + 1] * s2_ref[...]

    return pl.pallas_call(
        body,
        grid=(grid,),
        in_specs=[
            pl.BlockSpec((NC, _NBLK, 16), lambda i: (0, i, 0)),
            pl.BlockSpec((_NBLK, EMB), lambda i: (i, 0)),
        ],
        out_specs=pl.BlockSpec((_NBLK, EMB), lambda i: (i, 0)),
        out_shape=jax.ShapeDtypeStruct((NOP, EMB), jnp.float32),
    )(parts, s2)


def kernel(x_op, x_res, params, precedence_edges, requirement_edges):
    rq_src = requirement_edges[0]
    rq_dst = requirement_edges[1]
    pe_src = precedence_edges[0]
    pe_dst = precedence_edges[1]

    g_rq_src = _pad_idx(rq_src, 0)
    s_rq_dst = _pad_idx(rq_dst, NRES)
    g_rq_dst = _pad_idx(rq_dst, 0)
    s_rq_src = _pad_idx(rq_src, NOP)
    g_pe_src = _pad_idx(pe_src, 0)
    g_pe_dst = _pad_idx(pe_dst, 0)
    s_pe_dst = _pad_idx(pe_dst, NOP)

    z_res = jnp.zeros((R_RES, 16), jnp.float32)
    z_op = jnp.zeros((R_OP, 16), jnp.float32)

    lp0, lp1 = params["res_layers"]
    t16 = _tc_res_pre(x_res, lp0["W"], lp0["b"])
    parts = _sc_aggregate(t16, g_rq_src, s_rq_dst, z_res, R_RES)
    t16 = _tc_res_next(parts, lp1["W"], lp1["b"])
    parts = _sc_aggregate(t16, g_rq_src, s_rq_dst, z_res, R_RES)
    r, r16 = _tc_res_fin(parts)

    aggparts = _sc_aggregate(r16, g_rq_dst, s_rq_src, z_op, R_OP)

    o = x_op
    for lp in params["op_layers"]:
        p8, q8, s2 = _tc_node(o, aggparts, lp)
        pg, qg = _sc_gather2(p8, q8, g_pe_src, g_pe_dst)
        m16 = _tc_comb(pg.reshape(EP // 16, 128), qg.reshape(EP // 16, 128),
                       lp["combined"])
        eparts = _sc_scatter(m16, s_pe_dst, z_op, R_OP)
        o = _tc_fin(eparts, s2)

    return o, r


# speedup vs baseline: 1.6189x; 1.0262x over previous
import functools

import jax
import jax.numpy as jnp
from jax import lax
from jax.experimental import pallas as pl
from jax.experimental.pallas import tpu as pltpu
from jax.experimental.pallas import tpu_sc as plsc

NOP = 10000
NRES = 1000
E = 160000
EMB = 8

NC = 2
NS = 16
NW = NC * NS
CH = 128
EP = 163840
NCH = EP // (NW * CH)
PW = NCH * CH
R_OP = 10240
R_RES = 1024

_MESH = plsc.VectorSubcoreMesh(core_axis_name="c", subcore_axis_name="s")


def _pad_idx(idx, fill):
    pad = jnp.full((EP - E,), fill, jnp.int32)
    return jnp.concatenate([idx.astype(jnp.int32), pad]).reshape(NW, NCH, CH)


def _sc_aggregate(table16, gidx, sidx, zrows, nrows):
    rpt = nrows // NS

    def body(table_h, gidx_h, sidx_h, zeros_h, out_h, gidx_v, sidx_v, rows_v, acc_sh, sem):
        c = lax.axis_index("c")
        s = lax.axis_index("s")
        wid = s * NC + c
        pltpu.sync_copy(zeros_h.at[pl.ds(s * rpt, rpt)], acc_sh.at[pl.ds(s * rpt, rpt)])
        pltpu.sync_copy(gidx_h.at[wid], gidx_v)
        pltpu.sync_copy(sidx_h.at[wid], sidx_v)
        plsc.subcore_barrier()

        @pl.loop(0, NCH)
        def _fire(j):
            pltpu.async_copy(table_h.at[gidx_v.at[j]], rows_v.at[j], sem)

        @pl.loop(0, NCH)
        def _drain(j):
            pltpu.make_async_copy(table_h.at[gidx_v.at[j]], rows_v.at[j], sem).wait()
            pltpu.sync_copy(rows_v.at[j], acc_sh.at[sidx_v.at[j]], add=True)

        plsc.subcore_barrier()
        pltpu.sync_copy(acc_sh.at[pl.ds(s * rpt, rpt)], out_h.at[c, pl.ds(s * rpt, rpt)])

    f = pl.kernel(
        body,
        out_type=jax.ShapeDtypeStruct((NC, nrows, 16), jnp.float32),
        mesh=_MESH,
        compiler_params=pltpu.CompilerParams(use_tc_tiling_on_sc=False),
        scratch_types=[
            pltpu.VMEM((NCH, CH), jnp.int32),
            pltpu.VMEM((NCH, CH), jnp.int32),
            pltpu.VMEM((NCH, CH, 16), jnp.float32),
            pltpu.VMEM_SHARED((nrows, 16), jnp.float32),
            pltpu.SemaphoreType.DMA,
        ],
    )
    return f(table16, gidx, sidx, zrows)


def _sc_gather2(p8, q8, sidx, didx):
    def body(p_h, q_h, si_h, di_h, op_h, oq_h, si_v, di_v, rp_v, rq_v, semp, semq):
        c = lax.axis_index("c")
        s = lax.axis_index("s")
        wid = s * NC + c
        pltpu.sync_copy(si_h.at[wid], si_v)
        pltpu.sync_copy(di_h.at[wid], di_v)

        @pl.loop(0, NCH)
        def _fire(j):
            pltpu.async_copy(p_h.at[si_v.at[j]], rp_v.at[pl.ds(j * CH, CH)], semp)
            pltpu.async_copy(q_h.at[di_v.at[j]], rq_v.at[pl.ds(j * CH, CH)], semq)

        @pl.loop(0, NCH)
        def _drain(j):
            pltpu.make_async_copy(p_h.at[si_v.at[j]], rp_v.at[pl.ds(j * CH, CH)], semp).wait()
            pltpu.make_async_copy(q_h.at[di_v.at[j]], rq_v.at[pl.ds(j * CH, CH)], semq).wait()

        pltpu.sync_copy(rp_v, op_h.at[pl.ds(wid * PW, PW)])
        pltpu.sync_copy(rq_v, oq_h.at[pl.ds(wid * PW, PW)])

    f = pl.kernel(
        body,
        out_type=[
            jax.ShapeDtypeStruct((EP, EMB), jnp.float32),
            jax.ShapeDtypeStruct((EP, EMB), jnp.float32),
        ],
        mesh=_MESH,
        compiler_params=pltpu.CompilerParams(use_tc_tiling_on_sc=False),
        scratch_types=[
            pltpu.VMEM((NCH, CH), jnp.int32),
            pltpu.VMEM((NCH, CH), jnp.int32),
            pltpu.VMEM((PW, EMB), jnp.float32),
            pltpu.VMEM((PW, EMB), jnp.float32),
            pltpu.SemaphoreType.DMA,
            pltpu.SemaphoreType.DMA,
        ],
    )
    return f(p8, q8, sidx, didx)


def _sc_scatter(m16, sidx, zrows, nrows):
    rpt = nrows // NS

    def body(m_h, sidx_h, zeros_h, out_h, sidx_v, rows_v, acc_sh):
        c = lax.axis_index("c")
        s = lax.axis_index("s")
        wid = s * NC + c
        pltpu.sync_copy(zeros_h.at[pl.ds(s * rpt, rpt)], acc_sh.at[pl.ds(s * rpt, rpt)])
        pltpu.sync_copy(sidx_h.at[wid], sidx_v)
        pltpu.sync_copy(m_h.at[pl.ds(wid * PW, PW)], rows_v)
        plsc.subcore_barrier()

        @pl.loop(0, NCH)
        def _scat(j):
            pltpu.sync_copy(rows_v.at[pl.ds(j * CH, CH)], acc_sh.at[sidx_v.at[j]], add=True)

        plsc.subcore_barrier()
        pltpu.sync_copy(acc_sh.at[pl.ds(s * rpt, rpt)], out_h.at[c, pl.ds(s * rpt, rpt)])

    f = pl.kernel(
        body,
        out_type=jax.ShapeDtypeStruct((NC, nrows, 16), jnp.float32),
        mesh=_MESH,
        compiler_params=pltpu.CompilerParams(use_tc_tiling_on_sc=False),
        scratch_types=[
            pltpu.VMEM((NCH, CH), jnp.int32),
            pltpu.VMEM((PW, 16), jnp.float32),
            pltpu.VMEM_SHARED((nrows, 16), jnp.float32),
        ],
    )
    return f(m16, sidx, zrows)


def _dot(a, b):
    return jnp.dot(a.astype(jnp.bfloat16), b.astype(jnp.bfloat16),
                   preferred_element_type=jnp.float32)


def _elu(x):
    return jnp.where(x > 0, x, jnp.exp(jnp.minimum(x, 0.0)) - 1.0)


def _mlp3(x, w0, b0, w1, b1, w2, b2):
    h = _elu(_dot(x, w0[...]) + b0[...])
    h = _elu(_dot(h, w1[...]) + b1[...])
    return _dot(h, w2[...]) + b2[...]


def _with_count_col(t, count_val):
    n = t.shape[0]
    col = lax.broadcasted_iota(jnp.int32, (n, 16), 1)
    tt = jnp.concatenate([t, t], axis=1)
    return jnp.where(col < EMB, tt, jnp.where(col == EMB, count_val, 0.0))


def _mlp_flat(mlp):
    out = []
    for lin in mlp:
        out.append(lin["W"])
        out.append(lin["b"].reshape(1, -1))
    return out


def _full_spec(a):
    return pl.BlockSpec(a.shape, lambda *_: (0,) * a.ndim)


def _tc_res_pre(x, w, b):
    def body(x_ref, w_ref, b_ref, out_ref):
        t = _dot(x_ref[...], w_ref[...]) + b_ref[...]
        out_ref[...] = _with_count_col(t, 1.0)

    return pl.pallas_call(
        body,
        out_shape=jax.ShapeDtypeStruct((NRES, 16), jnp.float32),
    )(x, w, b.reshape(1, -1))


def _tc_res_next(parts, w, b):
    def body(p_ref, w_ref, b_ref, out_ref):
        p = p_ref[...]
        sums = (p[0] + p[1])[:NRES]
        r = sums[:, :EMB] / jnp.maximum(sums[:, EMB:EMB + 1], 1.0)
        t = _dot(r, w_ref[...]) + b_ref[...]
        out_ref[...] = _with_count_col(t, 1.0)

    return pl.pallas_call(
        body,
        out_shape=jax.ShapeDtypeStruct((NRES, 16), jnp.float32),
    )(parts, w, b.reshape(1, -1))


def _tc_res_fin(parts):
    def body(p_ref, r_ref, r16_ref):
        p = p_ref[...]
        sums = (p[0] + p[1])[:NRES]
        r = sums[:, :EMB] / jnp.maximum(sums[:, EMB:EMB + 1], 1.0)
        r_ref[...] = r
        r16_ref[...] = _with_count_col(r, 0.0)

    return pl.pallas_call(
        body,
        out_shape=[
            jax.ShapeDtypeStruct((NRES, EMB), jnp.float32),
            jax.ShapeDtypeStruct((NRES, 16), jnp.float32),
        ],
    )(parts)


_NBLK = 1000


def _tc_node(x, aggparts, lp):
    fi = x.shape[1]
    grid = NOP // _NBLK
    weights = (_mlp_flat(lp["predecessor"]) + _mlp_flat(lp["successor"])
               + _mlp_flat(lp["resources"]) + _mlp_flat(lp["same"]))

    def body(x_ref, agg_ref, *refs):
        w = refs[:24]
        p_ref, q_ref, s2_ref = refs[24:]
        x_v = x_ref[...]
        a = agg_ref[...]
        aggv = (a[0] + a[1])[:, :EMB]
        p_ref[...] = _mlp3(x_v, *w[0:6])
        q_ref[...] = _mlp3(aggv, *w[12:18]) + _mlp3(x_v, *w[6:12])
        s2_ref[...] = _mlp3(x_v, *w[18:24])

    in_specs = [
        pl.BlockSpec((_NBLK, fi), lambda i: (i, 0)),
        pl.BlockSpec((NC, _NBLK, 16), lambda i: (0, i, 0)),
    ] + [_full_spec(a) for a in weights]
    out_spec = pl.BlockSpec((_NBLK, EMB), lambda i: (i, 0))
    return pl.pallas_call(
        body,
        grid=(grid,),
        in_specs=in_specs,
        out_specs=[out_spec] * 3,
        out_shape=[jax.ShapeDtypeStruct((NOP, EMB), jnp.float32)] * 3,
    )(x, aggparts, *weights)


_EBLK = 4096


def _tc_comb(pg128, qg128, mlp):
    grid = EP // _EBLK

    w1, b1 = mlp[0]["W"], mlp[0]["b"]
    w2, b2 = mlp[1]["W"], mlp[1]["b"]
    w3, b3 = mlp[2]["W"], mlp[2]["b"]
    w1bd = jnp.zeros((128, 16 * 128), jnp.float32)
    for g in range(16):
        w1bd = w1bd.at[g * EMB:(g + 1) * EMB, g * 128:(g + 1) * 128].set(w1)
    b1t = jnp.tile(b1, 16).reshape(1, 16 * 128)

    def body(pg_ref, qg_ref, w1_ref, b1_ref, w2_ref, b2_ref, w3_ref, b3_ref, out_ref):
        h1 = _elu(_dot(pg_ref[...] + qg_ref[...], w1_ref[...]) + b1_ref[...])
        h1 = h1.reshape(_EBLK, 128)
        h2 = _elu(_dot(h1, w2_ref[...]) + b2_ref[...])
        m = _dot(h2, w3_ref[...]) + b3_ref[...]
        out_ref[...] = _with_count_col(m, 1.0)

    args = [w1bd, b1t, w2, b2.reshape(1, -1), w3, b3.reshape(1, -1)]
    return pl.pallas_call(
        body,
        grid=(grid,),
        in_specs=[pl.BlockSpec((_EBLK // 16, 128), lambda i: (i, 0)),
                  pl.BlockSpec((_EBLK // 16, 128), lambda i: (i, 0))]
        + [_full_spec(a) for a in args],
        out_specs=pl.BlockSpec((_EBLK, 16), lambda i: (i, 0)),
        out_shape=jax.ShapeDtypeStruct((EP, 16), jnp.float32),
    )(pg128, qg128, *args)


def _tc_fin(parts, s2):
    grid = NOP // _NBLK

    def body(p_ref, s2_ref, o_ref):
        p = p_ref[...]
        tot = p[0] + p[1]
        o_ref[...] = tot[:, :EMB] + tot[:, EMB:EMB + 1] * s2_ref[...]

    return pl.pallas_call(
        body,
        grid=(grid,),
        in_specs=[
            pl.BlockSpec((NC, _NBLK, 16), lambda i: (0, i, 0)),
            pl.BlockSpec((_NBLK, EMB), lambda i: (i, 0)),
        ],
        out_specs=pl.BlockSpec((_NBLK, EMB), lambda i: (i, 0)),
        out_shape=jax.ShapeDtypeStruct((NOP, EMB), jnp.float32),
    )(parts, s2)


def kernel(x_op, x_res, params, precedence_edges, requirement_edges):
    rq_src = requirement_edges[0]
    rq_dst = requirement_edges[1]
    pe_src = precedence_edges[0]
    pe_dst = precedence_edges[1]

    g_rq_src = _pad_idx(rq_src, 0)
    s_rq_dst = _pad_idx(rq_dst, NRES)
    g_rq_dst = _pad_idx(rq_dst, 0)
    s_rq_src = _pad_idx(rq_src, NOP)
    g_pe_src = _pad_idx(pe_src, 0)
    g_pe_dst = _pad_idx(pe_dst, 0)
    s_pe_dst = _pad_idx(pe_dst, NOP)

    z_res = jnp.zeros((R_RES, 16), jnp.float32)
    z_op = jnp.zeros((R_OP, 16), jnp.float32)

    lp0, lp1 = params["res_layers"]
    t16 = _tc_res_pre(x_res, lp0["W"], lp0["b"])
    parts = _sc_aggregate(t16, g_rq_src, s_rq_dst, z_res, R_RES)
    t16 = _tc_res_next(parts, lp1["W"], lp1["b"])
    parts = _sc_aggregate(t16, g_rq_src, s_rq_dst, z_res, R_RES)
    r, r16 = _tc_res_fin(parts)

    aggparts = _sc_aggregate(r16, g_rq_dst, s_rq_src, z_op, R_OP)

    o = x_op
    for lp in params["op_layers"]:
        p8, q8, s2 = _tc_node(o, aggparts, lp)
        pg, qg = _sc_gather2(p8, q8, g_pe_src, g_pe_dst)
        m16 = _tc_comb(pg.reshape(EP // 16, 128), qg.reshape(EP // 16, 128),
                       lp["combined"])
        eparts = _sc_scatter(m16, s_pe_dst, z_op, R_OP)
        o = _tc_fin(eparts, s2)

    return o, r


# speedup vs baseline: 1.8648x; 1.1519x over previous
import functools

import jax
import jax.numpy as jnp
from jax import lax
from jax.experimental import pallas as pl
from jax.experimental.pallas import tpu as pltpu
from jax.experimental.pallas import tpu_sc as plsc

NOP = 10000
NRES = 1000
E = 160000
EMB = 8

NC = 2
NS = 16
NW = NC * NS
CH = 128
EP = 163840
NCH = EP // (NW * CH)
PW = NCH * CH
R_OP = 10240
R_RES = 1024

_MESH = plsc.VectorSubcoreMesh(core_axis_name="c", subcore_axis_name="s")


def _pad_idx(idx, fill):
    pad = jnp.full((EP - E,), fill, jnp.int32)
    return jnp.concatenate([idx.astype(jnp.int32), pad]).reshape(NW, NCH, CH)


def _sc_aggregate(table16, gidx, sidx, zrows, nrows):
    rpt = nrows // NS

    def body(table_h, gidx_h, sidx_h, zeros_h, out_h, gidx_v, sidx_v, rows_v, acc_sh, sem):
        c = lax.axis_index("c")
        s = lax.axis_index("s")
        wid = s * NC + c
        pltpu.sync_copy(zeros_h.at[pl.ds(s * rpt, rpt)], acc_sh.at[pl.ds(s * rpt, rpt)])
        pltpu.sync_copy(gidx_h.at[wid], gidx_v)
        pltpu.sync_copy(sidx_h.at[wid], sidx_v)
        plsc.subcore_barrier()

        @pl.loop(0, NCH)
        def _fire(j):
            pltpu.async_copy(table_h.at[gidx_v.at[j]], rows_v.at[j], sem)

        @pl.loop(0, NCH)
        def _drain(j):
            pltpu.make_async_copy(table_h.at[gidx_v.at[j]], rows_v.at[j], sem).wait()
            pltpu.sync_copy(rows_v.at[j], acc_sh.at[sidx_v.at[j]], add=True)

        plsc.subcore_barrier()
        pltpu.sync_copy(acc_sh.at[pl.ds(s * rpt, rpt)], out_h.at[c, pl.ds(s * rpt, rpt)])

    f = pl.kernel(
        body,
        out_type=jax.ShapeDtypeStruct((NC, nrows, 16), jnp.float32),
        mesh=_MESH,
        compiler_params=pltpu.CompilerParams(use_tc_tiling_on_sc=False),
        scratch_types=[
            pltpu.VMEM((NCH, CH), jnp.int32),
            pltpu.VMEM((NCH, CH), jnp.int32),
            pltpu.VMEM((NCH, CH, 16), jnp.float32),
            pltpu.VMEM_SHARED((nrows, 16), jnp.float32),
            pltpu.SemaphoreType.DMA,
        ],
    )
    return f(table16, gidx, sidx, zrows)


def _sc_gather2(p8, q8, sidx, didx):
    def body(p_h, q_h, si_h, di_h, op_h, oq_h, si_v, di_v, rp_v, rq_v, semp, semq):
        c = lax.axis_index("c")
        s = lax.axis_index("s")
        wid = s * NC + c
        pltpu.sync_copy(si_h.at[wid], si_v)
        pltpu.sync_copy(di_h.at[wid], di_v)

        @pl.loop(0, NCH)
        def _fire(j):
            pltpu.async_copy(p_h.at[si_v.at[j]], rp_v.at[pl.ds(j * CH, CH)], semp)
            pltpu.async_copy(q_h.at[di_v.at[j]], rq_v.at[pl.ds(j * CH, CH)], semq)

        @pl.loop(0, NCH)
        def _drain(j):
            pltpu.make_async_copy(p_h.at[si_v.at[j]], rp_v.at[pl.ds(j * CH, CH)], semp).wait()
            pltpu.make_async_copy(q_h.at[di_v.at[j]], rq_v.at[pl.ds(j * CH, CH)], semq).wait()

        pltpu.sync_copy(rp_v, op_h.at[pl.ds(wid * PW, PW)])
        pltpu.sync_copy(rq_v, oq_h.at[pl.ds(wid * PW, PW)])

    f = pl.kernel(
        body,
        out_type=[
            jax.ShapeDtypeStruct((EP, EMB), jnp.float32),
            jax.ShapeDtypeStruct((EP, EMB), jnp.float32),
        ],
        mesh=_MESH,
        compiler_params=pltpu.CompilerParams(use_tc_tiling_on_sc=False),
        scratch_types=[
            pltpu.VMEM((NCH, CH), jnp.int32),
            pltpu.VMEM((NCH, CH), jnp.int32),
            pltpu.VMEM((PW, EMB), jnp.float32),
            pltpu.VMEM((PW, EMB), jnp.float32),
            pltpu.SemaphoreType.DMA,
            pltpu.SemaphoreType.DMA,
        ],
    )
    return f(p8, q8, sidx, didx)


def _sc_scatter(m16, sidx, zrows, nrows):
    rpt = nrows // NS

    def body(m_h, sidx_h, zeros_h, out_h, sidx_v, rows_v, acc_sh):
        c = lax.axis_index("c")
        s = lax.axis_index("s")
        wid = s * NC + c
        pltpu.sync_copy(zeros_h.at[pl.ds(s * rpt, rpt)], acc_sh.at[pl.ds(s * rpt, rpt)])
        pltpu.sync_copy(sidx_h.at[wid], sidx_v)
        pltpu.sync_copy(m_h.at[pl.ds(wid * PW, PW)], rows_v)
        plsc.subcore_barrier()

        @pl.loop(0, NCH)
        def _scat(j):
            pltpu.sync_copy(rows_v.at[pl.ds(j * CH, CH)], acc_sh.at[sidx_v.at[j]], add=True)

        plsc.subcore_barrier()
        pltpu.sync_copy(acc_sh.at[pl.ds(s * rpt, rpt)], out_h.at[c, pl.ds(s * rpt, rpt)])

    f = pl.kernel(
        body,
        out_type=jax.ShapeDtypeStruct((NC, nrows, 16), jnp.float32),
        mesh=_MESH,
        compiler_params=pltpu.CompilerParams(use_tc_tiling_on_sc=False),
        scratch_types=[
            pltpu.VMEM((NCH, CH), jnp.int32),
            pltpu.VMEM((PW, 16), jnp.float32),
            pltpu.VMEM_SHARED((nrows, 16), jnp.float32),
        ],
    )
    return f(m16, sidx, zrows)


def _dot(a, b):
    return jnp.dot(a.astype(jnp.bfloat16), b.astype(jnp.bfloat16),
                   preferred_element_type=jnp.float32)


def _elu(x):
    return jnp.where(x > 0, x, jnp.exp(jnp.minimum(x, 0.0)) - 1.0)


def _mlp3(x, w0, b0, w1, b1, w2, b2):
    h = _elu(_dot(x, w0[...]) + b0[...])
    h = _elu(_dot(h, w1[...]) + b1[...])
    return _dot(h, w2[...]) + b2[...]


def _with_count_col(t, count_val):
    n = t.shape[0]
    col = lax.broadcasted_iota(jnp.int32, (n, 16), 1)
    tt = jnp.concatenate([t, t], axis=1)
    return jnp.where(col < EMB, tt, jnp.where(col == EMB, count_val, 0.0))


def _mlp_flat(mlp):
    out = []
    for lin in mlp:
        out.append(lin["W"])
        out.append(lin["b"].reshape(1, -1))
    return out


def _full_spec(a):
    return pl.BlockSpec(a.shape, lambda *_: (0,) * a.ndim)


def _tc_res_pre(x, w, b):
    def body(x_ref, w_ref, b_ref, out_ref):
        t = _dot(x_ref[...], w_ref[...]) + b_ref[...]
        out_ref[...] = _with_count_col(t, 1.0)

    return pl.pallas_call(
        body,
        out_shape=jax.ShapeDtypeStruct((NRES, 16), jnp.float32),
    )(x, w, b.reshape(1, -1))


def _tc_res_next(parts, w, b):
    def body(p_ref, w_ref, b_ref, out_ref):
        p = p_ref[...]
        sums = (p[0] + p[1])[:NRES]
        r = sums[:, :EMB] / jnp.maximum(sums[:, EMB:EMB + 1], 1.0)
        t = _dot(r, w_ref[...]) + b_ref[...]
        out_ref[...] = _with_count_col(t, 1.0)

    return pl.pallas_call(
        body,
        out_shape=jax.ShapeDtypeStruct((NRES, 16), jnp.float32),
    )(parts, w, b.reshape(1, -1))


def _tc_res_fin(parts):
    def body(p_ref, r_ref, r16_ref):
        p = p_ref[...]
        sums = (p[0] + p[1])[:NRES]
        r = sums[:, :EMB] / jnp.maximum(sums[:, EMB:EMB + 1], 1.0)
        r_ref[...] = r
        r16_ref[...] = _with_count_col(r, 0.0)

    return pl.pallas_call(
        body,
        out_shape=[
            jax.ShapeDtypeStruct((NRES, EMB), jnp.float32),
            jax.ShapeDtypeStruct((NRES, 16), jnp.float32),
        ],
    )(parts)


_NBLK = 1000


def _tc_node(x, aggparts, lp):
    fi = x.shape[1]
    grid = NOP // _NBLK
    weights = (_mlp_flat(lp["predecessor"]) + _mlp_flat(lp["successor"])
               + _mlp_flat(lp["resources"]) + _mlp_flat(lp["same"]))

    def body(x_ref, agg_ref, *refs):
        w = refs[:24]
        p_ref, q_ref, s2_ref = refs[24:]
        x_v = x_ref[...]
        a = agg_ref[...]
        aggv = (a[0] + a[1])[:, :EMB]
        p_ref[...] = _mlp3(x_v, *w[0:6])
        q_ref[...] = _mlp3(aggv, *w[12:18]) + _mlp3(x_v, *w[6:12])
        s2_ref[...] = _mlp3(x_v, *w[18:24])

    in_specs = [
        pl.BlockSpec((_NBLK, fi), lambda i: (i, 0)),
        pl.BlockSpec((NC, _NBLK, 16), lambda i: (0, i, 0)),
    ] + [_full_spec(a) for a in weights]
    out_spec = pl.BlockSpec((_NBLK, EMB), lambda i: (i, 0))
    return pl.pallas_call(
        body,
        grid=(grid,),
        in_specs=in_specs,
        out_specs=[out_spec] * 3,
        out_shape=[jax.ShapeDtypeStruct((NOP, EMB), jnp.float32)] * 3,
    )(x, aggparts, *weights)


_EBLK = 4096


def _tc_comb(pg128, qg128, mlp):
    grid = EP // _EBLK

    w1, b1 = mlp[0]["W"], mlp[0]["b"]
    w2, b2 = mlp[1]["W"], mlp[1]["b"]
    w3, b3 = mlp[2]["W"], mlp[2]["b"]
    w1bd = jnp.zeros((128, 16 * 128), jnp.float32)
    for g in range(16):
        w1bd = w1bd.at[g * EMB:(g + 1) * EMB, g * 128:(g + 1) * 128].set(w1)
    b1t = jnp.tile(b1, 16).reshape(1, 16 * 128)

    w3e = jnp.concatenate([w3, jnp.zeros((128, 8), jnp.float32)], axis=1)
    w3bd = jnp.zeros((8 * 128, 128), jnp.float32)
    for g in range(8):
        w3bd = w3bd.at[g * 128:(g + 1) * 128, g * 16:(g + 1) * 16].set(w3e)
    b3e = jnp.concatenate([b3, jnp.ones((1,), jnp.float32),
                           jnp.zeros((7,), jnp.float32)])
    b3t = jnp.tile(b3e, 8).reshape(1, 128)

    def body(pg_ref, qg_ref, w1_ref, b1_ref, w2_ref, b2_ref, w3_ref, b3_ref, out_ref):
        h1 = _elu(_dot(pg_ref[...] + qg_ref[...], w1_ref[...]) + b1_ref[...])
        h1 = h1.reshape(_EBLK, 128)
        h2 = _elu(_dot(h1, w2_ref[...]) + b2_ref[...])
        h2 = h2.reshape(_EBLK // 8, 8 * 128)
        out_ref[...] = _dot(h2, w3_ref[...]) + b3_ref[...]

    args = [w1bd, b1t, w2, b2.reshape(1, -1), w3bd, b3t]
    return pl.pallas_call(
        body,
        grid=(grid,),
        in_specs=[pl.BlockSpec((_EBLK // 16, 128), lambda i: (i, 0)),
                  pl.BlockSpec((_EBLK // 16, 128), lambda i: (i, 0))]
        + [_full_spec(a) for a in args],
        out_specs=pl.BlockSpec((_EBLK // 8, 128), lambda i: (i, 0)),
        out_shape=jax.ShapeDtypeStruct((EP // 8, 128), jnp.float32),
    )(pg128, qg128, *args)


def _tc_fin(parts, s2):
    grid = NOP // _NBLK

    def body(p_ref, s2_ref, o_ref):
        p = p_ref[...]
        tot = p[0] + p[1]
        o_ref[...] = tot[:, :EMB] + tot[:, EMB:EMB + 1] * s2_ref[...]

    return pl.pallas_call(
        body,
        grid=(grid,),
        in_specs=[
            pl.BlockSpec((NC, _NBLK, 16), lambda i: (0, i, 0)),
            pl.BlockSpec((_NBLK, EMB), lambda i: (i, 0)),
        ],
        out_specs=pl.BlockSpec((_NBLK, EMB), lambda i: (i, 0)),
        out_shape=jax.ShapeDtypeStruct((NOP, EMB), jnp.float32),
    )(parts, s2)


def kernel(x_op, x_res, params, precedence_edges, requirement_edges):
    rq_src = requirement_edges[0]
    rq_dst = requirement_edges[1]
    pe_src = precedence_edges[0]
    pe_dst = precedence_edges[1]

    g_rq_src = _pad_idx(rq_src, 0)
    s_rq_dst = _pad_idx(rq_dst, NRES)
    g_rq_dst = _pad_idx(rq_dst, 0)
    s_rq_src = _pad_idx(rq_src, NOP)
    g_pe_src = _pad_idx(pe_src, 0)
    g_pe_dst = _pad_idx(pe_dst, 0)
    s_pe_dst = _pad_idx(pe_dst, NOP)

    z_res = jnp.zeros((R_RES, 16), jnp.float32)
    z_op = jnp.zeros((R_OP, 16), jnp.float32)

    lp0, lp1 = params["res_layers"]
    t16 = _tc_res_pre(x_res, lp0["W"], lp0["b"])
    parts = _sc_aggregate(t16, g_rq_src, s_rq_dst, z_res, R_RES)
    t16 = _tc_res_next(parts, lp1["W"], lp1["b"])
    parts = _sc_aggregate(t16, g_rq_src, s_rq_dst, z_res, R_RES)
    r, r16 = _tc_res_fin(parts)

    aggparts = _sc_aggregate(r16, g_rq_dst, s_rq_src, z_op, R_OP)

    o = x_op
    for lp in params["op_layers"]:
        p8, q8, s2 = _tc_node(o, aggparts, lp)
        pg, qg = _sc_gather2(p8, q8, g_pe_src, g_pe_dst)
        m128 = _tc_comb(pg.reshape(EP // 16, 128), qg.reshape(EP // 16, 128),
                        lp["combined"])
        eparts = _sc_scatter(m128.reshape(EP, 16), s_pe_dst, z_op, R_OP)
        o = _tc_fin(eparts, s2)

    return o, r


# speedup vs baseline: 1.8836x; 1.0101x over previous
import functools

import jax
import jax.numpy as jnp
from jax import lax
from jax.experimental import pallas as pl
from jax.experimental.pallas import tpu as pltpu
from jax.experimental.pallas import tpu_sc as plsc

NOP = 10000
NRES = 1000
E = 160000
EMB = 8

NC = 2
NS = 16
NW = NC * NS
CH = 128
EP = 163840
NCH = EP // (NW * CH)
PW = NCH * CH
R_OP = 10240
R_RES = 1024

_MESH = plsc.VectorSubcoreMesh(core_axis_name="c", subcore_axis_name="s")


def _pad_idx(idx, fill):
    pad = jnp.full((EP - E,), fill, jnp.int32)
    return jnp.concatenate([idx.astype(jnp.int32), pad]).reshape(NW, NCH, CH)


def _sc_aggregate(table16, gidx, sidx, zrows, nrows):
    rpt = nrows // NS

    def body(table_h, gidx_h, sidx_h, zeros_h, out_h, gidx_v, sidx_v, rows_v, acc_sh, sem):
        c = lax.axis_index("c")
        s = lax.axis_index("s")
        wid = s * NC + c
        pltpu.sync_copy(zeros_h.at[pl.ds(s * rpt, rpt)], acc_sh.at[pl.ds(s * rpt, rpt)])
        pltpu.sync_copy(gidx_h.at[wid], gidx_v)
        pltpu.sync_copy(sidx_h.at[wid], sidx_v)
        plsc.subcore_barrier()

        @pl.loop(0, NCH)
        def _fire(j):
            pltpu.async_copy(table_h.at[gidx_v.at[j]], rows_v.at[j], sem)

        @pl.loop(0, NCH)
        def _drain(j):
            pltpu.make_async_copy(table_h.at[gidx_v.at[j]], rows_v.at[j], sem).wait()
            pltpu.sync_copy(rows_v.at[j], acc_sh.at[sidx_v.at[j]], add=True)

        plsc.subcore_barrier()
        pltpu.sync_copy(acc_sh.at[pl.ds(s * rpt, rpt)], out_h.at[c, pl.ds(s * rpt, rpt)])

    f = pl.kernel(
        body,
        out_type=jax.ShapeDtypeStruct((NC, nrows, 16), jnp.float32),
        mesh=_MESH,
        compiler_params=pltpu.CompilerParams(use_tc_tiling_on_sc=False),
        scratch_types=[
            pltpu.VMEM((NCH, CH), jnp.int32),
            pltpu.VMEM((NCH, CH), jnp.int32),
            pltpu.VMEM((NCH, CH, 16), jnp.float32),
            pltpu.VMEM_SHARED((nrows, 16), jnp.float32),
            pltpu.SemaphoreType.DMA,
        ],
    )
    return f(table16, gidx, sidx, zrows)


def _sc_gather2(p8, q8, sidx, didx):
    def body(p_h, q_h, si_h, di_h, op_h, oq_h, si_v, di_v, rp_v, rq_v, semp, semq):
        c = lax.axis_index("c")
        s = lax.axis_index("s")
        wid = s * NC + c
        pltpu.sync_copy(si_h.at[wid], si_v)
        pltpu.sync_copy(di_h.at[wid], di_v)

        @pl.loop(0, NCH)
        def _fire(j):
            pltpu.async_copy(p_h.at[si_v.at[j]], rp_v.at[pl.ds(j * CH, CH)], semp)
            pltpu.async_copy(q_h.at[di_v.at[j]], rq_v.at[pl.ds(j * CH, CH)], semq)

        @pl.loop(0, NCH)
        def _drain(j):
            pltpu.make_async_copy(p_h.at[si_v.at[j]], rp_v.at[pl.ds(j * CH, CH)], semp).wait()
            pltpu.make_async_copy(q_h.at[di_v.at[j]], rq_v.at[pl.ds(j * CH, CH)], semq).wait()

        pltpu.sync_copy(rp_v, op_h.at[pl.ds(wid * PW, PW)])
        pltpu.sync_copy(rq_v, oq_h.at[pl.ds(wid * PW, PW)])

    f = pl.kernel(
        body,
        out_type=[
            jax.ShapeDtypeStruct((EP, EMB), jnp.float32),
            jax.ShapeDtypeStruct((EP, EMB), jnp.float32),
        ],
        mesh=_MESH,
        compiler_params=pltpu.CompilerParams(use_tc_tiling_on_sc=False),
        scratch_types=[
            pltpu.VMEM((NCH, CH), jnp.int32),
            pltpu.VMEM((NCH, CH), jnp.int32),
            pltpu.VMEM((PW, EMB), jnp.float32),
            pltpu.VMEM((PW, EMB), jnp.float32),
            pltpu.SemaphoreType.DMA,
            pltpu.SemaphoreType.DMA,
        ],
    )
    return f(p8, q8, sidx, didx)


def _sc_scatter(m16, sidx, zrows, nrows):
    rpt = nrows // NS

    def body(m_h, sidx_h, zeros_h, out_h, sidx_v, rows_v, acc_sh):
        c = lax.axis_index("c")
        s = lax.axis_index("s")
        wid = s * NC + c
        pltpu.sync_copy(zeros_h.at[pl.ds(s * rpt, rpt)], acc_sh.at[pl.ds(s * rpt, rpt)])
        pltpu.sync_copy(sidx_h.at[wid], sidx_v)
        pltpu.sync_copy(m_h.at[pl.ds(wid * PW, PW)], rows_v)
        plsc.subcore_barrier()

        @pl.loop(0, NCH)
        def _scat(j):
            pltpu.sync_copy(rows_v.at[pl.ds(j * CH, CH)], acc_sh.at[sidx_v.at[j]], add=True)

        plsc.subcore_barrier()
        pltpu.sync_copy(acc_sh.at[pl.ds(s * rpt, rpt)], out_h.at[c, pl.ds(s * rpt, rpt)])

    f = pl.kernel(
        body,
        out_type=jax.ShapeDtypeStruct((NC, nrows, 16), jnp.float32),
        mesh=_MESH,
        compiler_params=pltpu.CompilerParams(use_tc_tiling_on_sc=False),
        scratch_types=[
            pltpu.VMEM((NCH, CH), jnp.int32),
            pltpu.VMEM((PW, 16), jnp.float32),
            pltpu.VMEM_SHARED((nrows, 16), jnp.float32),
        ],
    )
    return f(m16, sidx, zrows)


def _dot(a, b):
    return jnp.dot(a.astype(jnp.bfloat16), b.astype(jnp.bfloat16),
                   preferred_element_type=jnp.float32)


def _elu(x):
    return jnp.where(x > 0, x, jnp.exp(jnp.minimum(x, 0.0)) - 1.0)


def _mlp3(x, w0, b0, w1, b1, w2, b2):
    h = _elu(_dot(x, w0[...]) + b0[...])
    h = _elu(_dot(h, w1[...]) + b1[...])
    return _dot(h, w2[...]) + b2[...]


def _with_count_col(t, count_val):
    n = t.shape[0]
    col = lax.broadcasted_iota(jnp.int32, (n, 16), 1)
    tt = jnp.concatenate([t, t], axis=1)
    return jnp.where(col < EMB, tt, jnp.where(col == EMB, count_val, 0.0))


def _mlp_flat(mlp):
    out = []
    for lin in mlp:
        out.append(lin["W"])
        out.append(lin["b"].reshape(1, -1))
    return out


def _full_spec(a):
    return pl.BlockSpec(a.shape, lambda *_: (0,) * a.ndim)


def _tc_res_pre(x, w, b):
    def body(x_ref, w_ref, b_ref, out_ref):
        t = _dot(x_ref[...], w_ref[...]) + b_ref[...]
        out_ref[...] = _with_count_col(t, 1.0)

    return pl.pallas_call(
        body,
        out_shape=jax.ShapeDtypeStruct((NRES, 16), jnp.float32),
    )(x, w, b.reshape(1, -1))


def _tc_res_next(parts, w, b):
    def body(p_ref, w_ref, b_ref, out_ref):
        p = p_ref[...]
        sums = (p[0] + p[1])[:NRES]
        r = sums[:, :EMB] / jnp.maximum(sums[:, EMB:EMB + 1], 1.0)
        t = _dot(r, w_ref[...]) + b_ref[...]
        out_ref[...] = _with_count_col(t, 1.0)

    return pl.pallas_call(
        body,
        out_shape=jax.ShapeDtypeStruct((NRES, 16), jnp.float32),
    )(parts, w, b.reshape(1, -1))


def _tc_res_fin(parts):
    def body(p_ref, r_ref, r16_ref):
        p = p_ref[...]
        sums = (p[0] + p[1])[:NRES]
        r = sums[:, :EMB] / jnp.maximum(sums[:, EMB:EMB + 1], 1.0)
        r_ref[...] = r
        r16_ref[...] = _with_count_col(r, 0.0)

    return pl.pallas_call(
        body,
        out_shape=[
            jax.ShapeDtypeStruct((NRES, EMB), jnp.float32),
            jax.ShapeDtypeStruct((NRES, 16), jnp.float32),
        ],
    )(parts)


_NBLK = 1000


def _tc_node(xsrc, aggparts, lp):
    from_parts = isinstance(xsrc, tuple)
    grid = NOP // _NBLK
    weights = (_mlp_flat(lp["predecessor"]) + _mlp_flat(lp["successor"])
               + _mlp_flat(lp["resources"]) + _mlp_flat(lp["same"]))

    def body(x_ref, s2p_ref, agg_ref, *refs):
        w = refs[:24]
        p_ref, q_ref, s2_ref = refs[24:]
        if from_parts:
            t = x_ref[...]
            tot = t[0] + t[1]
            x_v = tot[:, :EMB] + tot[:, EMB:EMB + 1] * s2p_ref[...]
        else:
            x_v = x_ref[...]
        a = agg_ref[...]
        aggv = (a[0] + a[1])[:, :EMB]
        p_ref[...] = _mlp3(x_v, *w[0:6])
        q_ref[...] = _mlp3(aggv, *w[12:18]) + _mlp3(x_v, *w[6:12])
        s2_ref[...] = _mlp3(x_v, *w[18:24])

    if from_parts:
        eparts, s2prev = xsrc
        xa = [eparts, s2prev]
        xs = [pl.BlockSpec((NC, _NBLK, 16), lambda i: (0, i, 0)),
              pl.BlockSpec((_NBLK, EMB), lambda i: (i, 0))]
    else:
        xa = [xsrc, jnp.zeros((8, EMB), jnp.float32)]
        xs = [pl.BlockSpec((_NBLK, xsrc.shape[1]), lambda i: (i, 0)),
              _full_spec(xa[1])]
    in_specs = xs + [
        pl.BlockSpec((NC, _NBLK, 16), lambda i: (0, i, 0)),
    ] + [_full_spec(a) for a in weights]
    out_spec = pl.BlockSpec((_NBLK, EMB), lambda i: (i, 0))
    return pl.pallas_call(
        body,
        grid=(grid,),
        in_specs=in_specs,
        out_specs=[out_spec] * 3,
        out_shape=[jax.ShapeDtypeStruct((NOP, EMB), jnp.float32)] * 3,
    )(*xa, aggparts, *weights)


_EBLK = 4096


def _tc_comb(pg128, qg128, mlp):
    grid = EP // _EBLK

    w1, b1 = mlp[0]["W"], mlp[0]["b"]
    w2, b2 = mlp[1]["W"], mlp[1]["b"]
    w3, b3 = mlp[2]["W"], mlp[2]["b"]
    w1bd = jnp.zeros((128, 16 * 128), jnp.float32)
    for g in range(16):
        w1bd = w1bd.at[g * EMB:(g + 1) * EMB, g * 128:(g + 1) * 128].set(w1)
    b1t = jnp.tile(b1, 16).reshape(1, 16 * 128)

    w3e = jnp.concatenate([w3, jnp.zeros((128, 8), jnp.float32)], axis=1)
    w3bd = jnp.zeros((8 * 128, 128), jnp.float32)
    for g in range(8):
        w3bd = w3bd.at[g * 128:(g + 1) * 128, g * 16:(g + 1) * 16].set(w3e)
    b3e = jnp.concatenate([b3, jnp.ones((1,), jnp.float32),
                           jnp.zeros((7,), jnp.float32)])
    b3t = jnp.tile(b3e, 8).reshape(1, 128)

    def body(pg_ref, qg_ref, w1_ref, b1_ref, w2_ref, b2_ref, w3_ref, b3_ref, out_ref):
        h1 = _elu(_dot(pg_ref[...] + qg_ref[...], w1_ref[...]) + b1_ref[...])
        h1 = h1.reshape(_EBLK, 128)
        h2 = _elu(_dot(h1, w2_ref[...]) + b2_ref[...])
        h2 = h2.reshape(_EBLK // 8, 8 * 128)
        out_ref[...] = _dot(h2, w3_ref[...]) + b3_ref[...]

    args = [w1bd, b1t, w2, b2.reshape(1, -1), w3bd, b3t]
    return pl.pallas_call(
        body,
        grid=(grid,),
        in_specs=[pl.BlockSpec((_EBLK // 16, 128), lambda i: (i, 0)),
                  pl.BlockSpec((_EBLK // 16, 128), lambda i: (i, 0))]
        + [_full_spec(a) for a in args],
        out_specs=pl.BlockSpec((_EBLK // 8, 128), lambda i: (i, 0)),
        out_shape=jax.ShapeDtypeStruct((EP // 8, 128), jnp.float32),
    )(pg128, qg128, *args)


def _tc_fin(parts, s2):
    grid = NOP // _NBLK

    def body(p_ref, s2_ref, o_ref):
        p = p_ref[...]
        tot = p[0] + p[1]
        o_ref[...] = tot[:, :EMB] + tot[:, EMB:EMB + 1] * s2_ref[...]

    return pl.pallas_call(
        body,
        grid=(grid,),
        in_specs=[
            pl.BlockSpec((NC, _NBLK, 16), lambda i: (0, i, 0)),
            pl.BlockSpec((_NBLK, EMB), lambda i: (i, 0)),
        ],
        out_specs=pl.BlockSpec((_NBLK, EMB), lambda i: (i, 0)),
        out_shape=jax.ShapeDtypeStruct((NOP, EMB), jnp.float32),
    )(parts, s2)


def kernel(x_op, x_res, params, precedence_edges, requirement_edges):
    rq_src = requirement_edges[0]
    rq_dst = requirement_edges[1]
    pe_src = precedence_edges[0]
    pe_dst = precedence_edges[1]

    g_rq_src = _pad_idx(rq_src, 0)
    s_rq_dst = _pad_idx(rq_dst, NRES)
    g_rq_dst = _pad_idx(rq_dst, 0)
    s_rq_src = _pad_idx(rq_src, NOP)
    g_pe_src = _pad_idx(pe_src, 0)
    g_pe_dst = _pad_idx(pe_dst, 0)
    s_pe_dst = _pad_idx(pe_dst, NOP)

    z_res = jnp.zeros((R_RES, 16), jnp.float32)
    z_op = jnp.zeros((R_OP, 16), jnp.float32)

    lp0, lp1 = params["res_layers"]
    t16 = _tc_res_pre(x_res, lp0["W"], lp0["b"])
    parts = _sc_aggregate(t16, g_rq_src, s_rq_dst, z_res, R_RES)
    t16 = _tc_res_next(parts, lp1["W"], lp1["b"])
    parts = _sc_aggregate(t16, g_rq_src, s_rq_dst, z_res, R_RES)
    r, r16 = _tc_res_fin(parts)

    aggparts = _sc_aggregate(r16, g_rq_dst, s_rq_src, z_op, R_OP)

    xsrc = x_op
    for lp in params["op_layers"]:
        p8, q8, s2 = _tc_node(xsrc, aggparts, lp)
        pg, qg = _sc_gather2(p8, q8, g_pe_src, g_pe_dst)
        m128 = _tc_comb(pg.reshape(EP // 16, 128), qg.reshape(EP // 16, 128),
                        lp["combined"])
        eparts = _sc_scatter(m128.reshape(EP, 16), s_pe_dst, z_op, R_OP)
        xsrc = (eparts, s2)

    o = _tc_fin(*xsrc)
    return o, r


# speedup vs baseline: 1.9377x; 1.0287x over previous
import functools

import jax
import jax.numpy as jnp
from jax import lax
from jax.experimental import pallas as pl
from jax.experimental.pallas import tpu as pltpu
from jax.experimental.pallas import tpu_sc as plsc

NOP = 10000
NRES = 1000
E = 160000
EMB = 8

NC = 2
NS = 16
NW = NC * NS
CH = 128
EP = 163840
NCH = EP // (NW * CH)
PW = NCH * CH
R_OP = 10240
R_RES = 1024

_MESH = plsc.VectorSubcoreMesh(core_axis_name="c", subcore_axis_name="s")


def _pad_idx(idx, fill):
    pad = jnp.full((EP - E,), fill, jnp.int32)
    return jnp.concatenate([idx.astype(jnp.int32), pad]).reshape(NW, NCH, CH)


def _sc_aggregate(table16, gidx, sidx, zrows, nrows):
    rpt = nrows // NS

    def body(table_h, gidx_h, sidx_h, zeros_h, out_h, gidx_v, sidx_v, rows_v, acc_sh, sem):
        c = lax.axis_index("c")
        s = lax.axis_index("s")
        wid = s * NC + c
        pltpu.sync_copy(zeros_h.at[pl.ds(s * rpt, rpt)], acc_sh.at[pl.ds(s * rpt, rpt)])
        pltpu.sync_copy(gidx_h.at[wid], gidx_v)
        pltpu.sync_copy(sidx_h.at[wid], sidx_v)
        plsc.subcore_barrier()

        @pl.loop(0, NCH)
        def _fire(j):
            pltpu.async_copy(table_h.at[gidx_v.at[j]], rows_v.at[j], sem)

        @pl.loop(0, NCH)
        def _drain(j):
            pltpu.make_async_copy(table_h.at[gidx_v.at[j]], rows_v.at[j], sem).wait()
            pltpu.sync_copy(rows_v.at[j], acc_sh.at[sidx_v.at[j]], add=True)

        plsc.subcore_barrier()
        pltpu.sync_copy(acc_sh.at[pl.ds(s * rpt, rpt)], out_h.at[c, pl.ds(s * rpt, rpt)])

    f = pl.kernel(
        body,
        out_type=jax.ShapeDtypeStruct((NC, nrows, 16), jnp.float32),
        mesh=_MESH,
        compiler_params=pltpu.CompilerParams(use_tc_tiling_on_sc=False),
        scratch_types=[
            pltpu.VMEM((NCH, CH), jnp.int32),
            pltpu.VMEM((NCH, CH), jnp.int32),
            pltpu.VMEM((NCH, CH, 16), jnp.float32),
            pltpu.VMEM_SHARED((nrows, 16), jnp.float32),
            pltpu.SemaphoreType.DMA,
        ],
    )
    return f(table16, gidx, sidx, zrows)


def _sc_gather2(p8, q8, sidx, didx):
    def body(p_h, q_h, si_h, di_h, op_h, oq_h, si_v, di_v, rp_v, rq_v, semp, semq):
        c = lax.axis_index("c")
        s = lax.axis_index("s")
        wid = s * NC + c
        pltpu.sync_copy(si_h.at[wid], si_v)
        pltpu.sync_copy(di_h.at[wid], di_v)

        @pl.loop(0, NCH)
        def _fire(j):
            pltpu.async_copy(p_h.at[si_v.at[j]], rp_v.at[pl.ds(j * CH, CH)], semp)
            pltpu.async_copy(q_h.at[di_v.at[j]], rq_v.at[pl.ds(j * CH, CH)], semq)

        @pl.loop(0, NCH)
        def _drain(j):
            pltpu.make_async_copy(p_h.at[si_v.at[j]], rp_v.at[pl.ds(j * CH, CH)], semp).wait()
            pltpu.make_async_copy(q_h.at[di_v.at[j]], rq_v.at[pl.ds(j * CH, CH)], semq).wait()

        pltpu.sync_copy(rp_v, op_h.at[pl.ds(wid * PW, PW)])
        pltpu.sync_copy(rq_v, oq_h.at[pl.ds(wid * PW, PW)])

    f = pl.kernel(
        body,
        out_type=[
            jax.ShapeDtypeStruct((EP, EMB), jnp.float32),
            jax.ShapeDtypeStruct((EP, EMB), jnp.float32),
        ],
        mesh=_MESH,
        compiler_params=pltpu.CompilerParams(use_tc_tiling_on_sc=False),
        scratch_types=[
            pltpu.VMEM((NCH, CH), jnp.int32),
            pltpu.VMEM((NCH, CH), jnp.int32),
            pltpu.VMEM((PW, EMB), jnp.float32),
            pltpu.VMEM((PW, EMB), jnp.float32),
            pltpu.SemaphoreType.DMA,
            pltpu.SemaphoreType.DMA,
        ],
    )
    return f(p8, q8, sidx, didx)


def _sc_scatter(m16, sidx, zrows, nrows):
    rpt = nrows // NS

    def body(m_h, sidx_h, zeros_h, out_h, sidx_v, rows_v, acc_sh):
        c = lax.axis_index("c")
        s = lax.axis_index("s")
        wid = s * NC + c
        pltpu.sync_copy(zeros_h.at[pl.ds(s * rpt, rpt)], acc_sh.at[pl.ds(s * rpt, rpt)])
        pltpu.sync_copy(sidx_h.at[wid], sidx_v)
        pltpu.sync_copy(m_h.at[pl.ds(wid * PW, PW)], rows_v)
        plsc.subcore_barrier()

        @pl.loop(0, NCH)
        def _scat(j):
            pltpu.sync_copy(rows_v.at[pl.ds(j * CH, CH)], acc_sh.at[sidx_v.at[j]], add=True)

        plsc.subcore_barrier()
        pltpu.sync_copy(acc_sh.at[pl.ds(s * rpt, rpt)], out_h.at[c, pl.ds(s * rpt, rpt)])

    f = pl.kernel(
        body,
        out_type=jax.ShapeDtypeStruct((NC, nrows, 16), jnp.float32),
        mesh=_MESH,
        compiler_params=pltpu.CompilerParams(use_tc_tiling_on_sc=False),
        scratch_types=[
            pltpu.VMEM((NCH, CH), jnp.int32),
            pltpu.VMEM((PW, 16), jnp.float32),
            pltpu.VMEM_SHARED((nrows, 16), jnp.float32),
        ],
    )
    return f(m16, sidx, zrows)


def _dot(a, b):
    return jnp.dot(a.astype(jnp.bfloat16), b.astype(jnp.bfloat16),
                   preferred_element_type=jnp.float32)


def _elu(x):
    return jnp.where(x > 0, x, jnp.exp(jnp.minimum(x, 0.0)) - 1.0)


def _mlp3(x, w0, b0, w1, b1, w2, b2):
    h = _elu(_dot(x, w0[...]) + b0[...])
    h = _elu(_dot(h, w1[...]) + b1[...])
    return _dot(h, w2[...]) + b2[...]


def _with_count_col(t, count_val):
    n = t.shape[0]
    col = lax.broadcasted_iota(jnp.int32, (n, 16), 1)
    tt = jnp.concatenate([t, t], axis=1)
    return jnp.where(col < EMB, tt, jnp.where(col == EMB, count_val, 0.0))


def _mlp_flat(mlp):
    out = []
    for lin in mlp:
        out.append(lin["W"])
        out.append(lin["b"].reshape(1, -1))
    return out


def _full_spec(a):
    return pl.BlockSpec(a.shape, lambda *_: (0,) * a.ndim)


def _tc_res_pre(x, w, b):
    def body(x_ref, w_ref, b_ref, out_ref):
        t = _dot(x_ref[...], w_ref[...]) + b_ref[...]
        out_ref[...] = _with_count_col(t, 1.0)

    return pl.pallas_call(
        body,
        out_shape=jax.ShapeDtypeStruct((NRES, 16), jnp.float32),
    )(x, w, b.reshape(1, -1))


def _tc_res_next(parts, w, b):
    def body(p_ref, w_ref, b_ref, out_ref):
        p = p_ref[...]
        sums = (p[0] + p[1])[:NRES]
        r = sums[:, :EMB] / jnp.maximum(sums[:, EMB:EMB + 1], 1.0)
        t = _dot(r, w_ref[...]) + b_ref[...]
        out_ref[...] = _with_count_col(t, 1.0)

    return pl.pallas_call(
        body,
        out_shape=jax.ShapeDtypeStruct((NRES, 16), jnp.float32),
    )(parts, w, b.reshape(1, -1))


def _tc_res_fin(parts):
    def body(p_ref, r_ref, r16_ref):
        p = p_ref[...]
        sums = (p[0] + p[1])[:NRES]
        r = sums[:, :EMB] / jnp.maximum(sums[:, EMB:EMB + 1], 1.0)
        r_ref[...] = r
        r16_ref[...] = _with_count_col(r, 0.0)

    return pl.pallas_call(
        body,
        out_shape=[
            jax.ShapeDtypeStruct((NRES, EMB), jnp.float32),
            jax.ShapeDtypeStruct((NRES, 16), jnp.float32),
        ],
    )(parts)


_NBLK = 1000


def _tc_node(xsrc, aggparts, lp):
    from_parts = isinstance(xsrc, tuple)
    grid = NOP // _NBLK
    weights = (_mlp_flat(lp["predecessor"]) + _mlp_flat(lp["successor"])
               + _mlp_flat(lp["resources"]) + _mlp_flat(lp["same"]))

    def body(x_ref, s2p_ref, agg_ref, *refs):
        w = refs[:24]
        p_ref, q_ref, s2_ref = refs[24:]
        if from_parts:
            t = x_ref[...]
            tot = t[0] + t[1]
            x_v = tot[:, :EMB] + tot[:, EMB:EMB + 1] * s2p_ref[...]
        else:
            x_v = x_ref[...]
        a = agg_ref[...]
        aggv = (a[0] + a[1])[:, :EMB]
        p_ref[...] = _mlp3(x_v, *w[0:6])
        q_ref[...] = _mlp3(aggv, *w[12:18]) + _mlp3(x_v, *w[6:12])
        s2_ref[...] = _mlp3(x_v, *w[18:24])

    if from_parts:
        eparts, s2prev = xsrc
        xa = [eparts, s2prev]
        xs = [pl.BlockSpec((NC, _NBLK, 16), lambda i: (0, i, 0)),
              pl.BlockSpec((_NBLK, EMB), lambda i: (i, 0))]
    else:
        xa = [xsrc, jnp.zeros((8, EMB), jnp.float32)]
        xs = [pl.BlockSpec((_NBLK, xsrc.shape[1]), lambda i: (i, 0)),
              _full_spec(xa[1])]
    in_specs = xs + [
        pl.BlockSpec((NC, _NBLK, 16), lambda i: (0, i, 0)),
    ] + [_full_spec(a) for a in weights]
    out_spec = pl.BlockSpec((_NBLK, EMB), lambda i: (i, 0))
    return pl.pallas_call(
        body,
        grid=(grid,),
        in_specs=in_specs,
        out_specs=[out_spec] * 3,
        out_shape=[jax.ShapeDtypeStruct((NOP, EMB), jnp.float32)] * 3,
    )(*xa, aggparts, *weights)


_EBLK = 8192


def _tc_comb(pg128, qg128, mlp):
    grid = EP // _EBLK

    w1, b1 = mlp[0]["W"], mlp[0]["b"]
    w2, b2 = mlp[1]["W"], mlp[1]["b"]
    w3, b3 = mlp[2]["W"], mlp[2]["b"]
    w1bd = jnp.zeros((128, 16 * 128), jnp.float32)
    for g in range(16):
        w1bd = w1bd.at[g * EMB:(g + 1) * EMB, g * 128:(g + 1) * 128].set(w1)
    b1t = jnp.tile(b1, 16).reshape(1, 16 * 128)

    w3e = jnp.concatenate([w3, jnp.zeros((128, 8), jnp.float32)], axis=1)
    w3bd = jnp.zeros((8 * 128, 128), jnp.float32)
    for g in range(8):
        w3bd = w3bd.at[g * 128:(g + 1) * 128, g * 16:(g + 1) * 16].set(w3e)
    b3e = jnp.concatenate([b3, jnp.ones((1,), jnp.float32),
                           jnp.zeros((7,), jnp.float32)])
    b3t = jnp.tile(b3e, 8).reshape(1, 128)

    def body(pg_ref, qg_ref, w1_ref, b1_ref, w2_ref, b2_ref, w3_ref, b3_ref, out_ref):
        h1 = _elu(_dot(pg_ref[...] + qg_ref[...], w1_ref[...]) + b1_ref[...])
        h1 = h1.reshape(_EBLK, 128)
        h2 = _elu(_dot(h1, w2_ref[...]) + b2_ref[...])
        h2 = h2.reshape(_EBLK // 8, 8 * 128)
        out_ref[...] = _dot(h2, w3_ref[...]) + b3_ref[...]

    args = [w1bd, b1t, w2, b2.reshape(1, -1), w3bd, b3t]
    return pl.pallas_call(
        body,
        grid=(grid,),
        in_specs=[pl.BlockSpec((_EBLK // 16, 128), lambda i: (i, 0)),
                  pl.BlockSpec((_EBLK // 16, 128), lambda i: (i, 0))]
        + [_full_spec(a) for a in args],
        out_specs=pl.BlockSpec((_EBLK // 8, 128), lambda i: (i, 0)),
        out_shape=jax.ShapeDtypeStruct((EP // 8, 128), jnp.float32),
    )(pg128, qg128, *args)


def _tc_fin(parts, s2):
    grid = NOP // _NBLK

    def body(p_ref, s2_ref, o_ref):
        p = p_ref[...]
        tot = p[0] + p[1]
        o_ref[...] = tot[:, :EMB] + tot[:, EMB:EMB + 1] * s2_ref[...]

    return pl.pallas_call(
        body,
        grid=(grid,),
        in_specs=[
            pl.BlockSpec((NC, _NBLK, 16), lambda i: (0, i, 0)),
            pl.BlockSpec((_NBLK, EMB), lambda i: (i, 0)),
        ],
        out_specs=pl.BlockSpec((_NBLK, EMB), lambda i: (i, 0)),
        out_shape=jax.ShapeDtypeStruct((NOP, EMB), jnp.float32),
    )(parts, s2)


def kernel(x_op, x_res, params, precedence_edges, requirement_edges):
    rq_src = requirement_edges[0]
    rq_dst = requirement_edges[1]
    pe_src = precedence_edges[0]
    pe_dst = precedence_edges[1]

    g_rq_src = _pad_idx(rq_src, 0)
    s_rq_dst = _pad_idx(rq_dst, NRES)
    g_rq_dst = _pad_idx(rq_dst, 0)
    s_rq_src = _pad_idx(rq_src, NOP)
    g_pe_src = _pad_idx(pe_src, 0)
    g_pe_dst = _pad_idx(pe_dst, 0)
    s_pe_dst = _pad_idx(pe_dst, NOP)

    z_res = jnp.zeros((R_RES, 16), jnp.float32)
    z_op = jnp.zeros((R_OP, 16), jnp.float32)

    lp0, lp1 = params["res_layers"]
    t16 = _tc_res_pre(x_res, lp0["W"], lp0["b"])
    parts = _sc_aggregate(t16, g_rq_src, s_rq_dst, z_res, R_RES)
    t16 = _tc_res_next(parts, lp1["W"], lp1["b"])
    parts = _sc_aggregate(t16, g_rq_src, s_rq_dst, z_res, R_RES)
    r, r16 = _tc_res_fin(parts)

    aggparts = _sc_aggregate(r16, g_rq_dst, s_rq_src, z_op, R_OP)

    xsrc = x_op
    for lp in params["op_layers"]:
        p8, q8, s2 = _tc_node(xsrc, aggparts, lp)
        pg, qg = _sc_gather2(p8, q8, g_pe_src, g_pe_dst)
        m128 = _tc_comb(pg.reshape(EP // 16, 128), qg.reshape(EP // 16, 128),
                        lp["combined"])
        eparts = _sc_scatter(m128.reshape(EP, 16), s_pe_dst, z_op, R_OP)
        xsrc = (eparts, s2)

    o = _tc_fin(*xsrc)
    return o, r
